# Initial kernel scaffold; baseline (speedup 1.0000x reference)
import jax
import jax.numpy as jnp
from jax.experimental import pallas as pl


def kernel(tokens, logprobs, num_tokens, local_seq_ids, new_tokens, new_log_probs, num_new_tokens):
    raise NotImplementedError("write your pallas kernel here")


# speedup vs baseline: 153.3176x; 153.3176x over previous
import functools

import jax
import jax.numpy as jnp
from jax import lax
from jax.experimental import pallas as pl
from jax.experimental.pallas import tpu as pltpu
from jax.experimental.pallas import tpu_sc as plsc

MAX_SEQS = 1024
MAX_TOKENS = 8192
NUM_NEW = 1024

_GROUPS = NUM_NEW // 16
_CHUNKS = NUM_NEW // 128


def _sc_body(num_tokens_hbm, sids_hbm, ntok_hbm, nlp_hbm, tok_flat, lp_flat,
             cnt_out_hbm, sids_v, cnt_v, idx_v, ntok_v, nlp_v, sem):
  @pl.when((lax.axis_index("c") == 0) & (lax.axis_index("s") == 0))
  def _():
    pltpu.sync_copy(sids_hbm, sids_v)
    pltpu.sync_copy(num_tokens_hbm, cnt_v)
    pltpu.sync_copy(ntok_hbm, ntok_v)
    pltpu.sync_copy(nlp_hbm, nlp_v)

    @pl.loop(0, _GROUPS)
    def _(g):
      v = sids_v[g]
      r, is_last = plsc.scan_count(v)
      base = plsc.load_gather(cnt_v, [v])
      pos = base + r - 1
      plsc.store_scatter(cnt_v, [v], pos + 1, mask=is_last)
      flat = v * MAX_TOKENS + pos
      row = g // 8
      col = (g % 8) * 16
      idx_v[row, pl.ds(col, 16)] = flat

    pltpu.sync_copy(cnt_v, cnt_out_hbm)

    copies = []
    for c in range(_CHUNKS):
      copies.append(
          pltpu.async_copy(ntok_v.at[c], tok_flat.at[idx_v.at[c]], sem))
      copies.append(
          pltpu.async_copy(nlp_v.at[c], lp_flat.at[idx_v.at[c]], sem))
    for cp in copies:
      cp.wait()


def _update(num_tokens, sids, ntok, nlp, tok_ref, lp_ref):
  mesh = plsc.VectorSubcoreMesh(core_axis_name="c", subcore_axis_name="s")
  return pl.kernel(
      _sc_body,
      out_type=jax.ShapeDtypeStruct((MAX_SEQS,), jnp.int32),
      mesh=mesh,
      compiler_params=pltpu.CompilerParams(needs_layout_passes=False),
      scratch_types=[
          pltpu.VMEM((_GROUPS, 16), jnp.int32),
          pltpu.VMEM((MAX_SEQS,), jnp.int32),
          pltpu.VMEM((_CHUNKS, 128), jnp.int32),
          pltpu.VMEM((_CHUNKS, 128), jnp.int32),
          pltpu.VMEM((_CHUNKS, 128), jnp.float32),
          pltpu.SemaphoreType.DMA,
      ],
  )(num_tokens, sids, ntok, nlp, tok_ref, lp_ref)


@jax.jit
def _kernel_impl(tokens, logprobs, num_tokens, local_seq_ids, new_tokens,
                 new_log_probs):
  tok_ref = jax.new_ref(tokens.reshape(-1))
  lp_ref = jax.new_ref(logprobs.reshape(-1))
  cnt = _update(
      num_tokens,
      local_seq_ids.reshape(_GROUPS, 16),
      new_tokens.reshape(_CHUNKS, 128),
      new_log_probs.reshape(_CHUNKS, 128),
      tok_ref,
      lp_ref,
  )
  return (
      tok_ref[...].reshape(MAX_SEQS, MAX_TOKENS),
      lp_ref[...].reshape(MAX_SEQS, MAX_TOKENS),
      cnt,
  )


def kernel(tokens, logprobs, num_tokens, local_seq_ids, new_tokens,
           new_log_probs, num_new_tokens):
  del num_new_tokens
  return _kernel_impl(tokens, logprobs, num_tokens, local_seq_ids, new_tokens,
                      new_log_probs)
